# baseline (speedup 1.0000x reference)
import functools

import jax
import jax.numpy as jnp
from jax import lax
from jax.experimental import pallas as pl
from jax.experimental.pallas import tpu as pltpu
from jax.experimental.pallas import tpu_sc as plsc

NC = 2
NS = 16
NW = NC * NS
CHUNK = 128
L = 16


def _iota16():
    return lax.iota(jnp.int32, L)


@functools.lru_cache(maxsize=None)
def _make_phase_a(V, D):
    n_tc = (V + CHUNK - 1) // CHUNK
    n_per_tile = (n_tc + NS - 1) // NS
    ri = n_tc * (CHUNK // 4)
    mesh = plsc.VectorSubcoreMesh(core_axis_name="c", subcore_axis_name="s")

    @functools.partial(
        pl.kernel,
        mesh=mesh,
        out_type=(jax.ShapeDtypeStruct((ri, 128), jnp.float32),
                  jax.ShapeDtypeStruct((ri, 128), jnp.float32)),
        scratch_types=[
            pltpu.VMEM((D, CHUNK), jnp.float32),
            pltpu.VMEM((CHUNK // 4, 128), jnp.float32),
            pltpu.SemaphoreType.DMA,
        ],
        compiler_params=pltpu.CompilerParams(disable_bounds_checks=True, needs_layout_passes=False),
    )
    def ka(ut3_hbm, it3_hbm, iu_hbm, ii_hbm, blk_v, outb_v, sem):
        cid = lax.axis_index("c")
        sid = lax.axis_index("s")

        def run(t3, inter):
            @pl.loop(0, n_per_tile)
            def _(t):
                tc = sid + t * NS

                @pl.when(tc < n_tc)
                def _():
                    col0 = pl.multiple_of(tc * CHUNK, CHUNK)
                    cps = [pltpu.async_copy(
                        t3.at[tr, :, pl.ds(col0, CHUNK)],
                        blk_v.at[pl.ds(8 * tr, 8)], sem)
                        for tr in range(D // 8)]
                    for cp in cps:
                        cp.wait()
                    rowbase = _iota16() // 4
                    colbase = (_iota16() % 4) * D
                    for c in range(D):
                        for m in range(CHUNK // L):
                            x = blk_v[c, pl.ds(L * m, L)]
                            plsc.store_scatter(
                                outb_v, [4 * m + rowbase, colbase + c], x)
                    row0 = pl.multiple_of(tc * (CHUNK // 4), CHUNK // 4)
                    pltpu.sync_copy(outb_v, inter.at[pl.ds(row0, CHUNK // 4)])

        @pl.when(cid == 0)
        def _():
            run(ut3_hbm, iu_hbm)

        @pl.when(cid == 1)
        def _():
            run(it3_hbm, ii_hbm)

    return ka


@functools.lru_cache(maxsize=None)
def _make_phase_b(B, D, ri):
    b_per_w = B // NW
    n_chunks = b_per_w // CHUNK
    mesh = plsc.VectorSubcoreMesh(core_axis_name="c", subcore_axis_name="s")

    @functools.partial(
        pl.kernel,
        mesh=mesh,
        out_type=jax.ShapeDtypeStruct((2 * D, B), jnp.float32),
        scratch_types=[
            pltpu.VMEM((b_per_w,), jnp.int32),
            pltpu.VMEM((b_per_w,), jnp.int32),
            pltpu.VMEM((b_per_w,), jnp.int32),
            pltpu.VMEM((b_per_w, 128), jnp.float32),
            pltpu.VMEM((2 * D, b_per_w), jnp.float32),
            pltpu.SemaphoreType.DMA,
        ],
        compiler_params=pltpu.CompilerParams(disable_bounds_checks=True, needs_layout_passes=False),
    )
    def kb(iu_hbm, ii_hbm, uidx_hbm, iidx_hbm, out_hbm,
           uidx_v, iidx_v, gr_v, rows_v, outb_v, sem):
        cid = lax.axis_index("c")
        sid = lax.axis_index("s")
        w = sid * NC + cid
        base = w * b_per_w
        pltpu.sync_copy(uidx_hbm.at[pl.ds(base, b_per_w)], uidx_v)
        pltpu.sync_copy(iidx_hbm.at[pl.ds(base, b_per_w)], iidx_v)

        def dotable(inter, idx_v, d_off):
            for v in range(b_per_w // L):
                x = idx_v[pl.ds(L * v, L)]
                gr_v[pl.ds(L * v, L)] = lax.shift_right_logical(x, 2)
            cps = []
            for j in range(n_chunks):
                cps.append(pltpu.async_copy(
                    inter.at[gr_v.at[pl.ds(j * CHUNK, CHUNK)]],
                    rows_v.at[pl.ds(j * CHUNK, CHUNK)], sem))
            for cp in cps:
                cp.wait()
            for v in range(b_per_w // L):
                x = idx_v[pl.ds(L * v, L)]
                colbase = (x & 3) * D
                rows16 = L * v + _iota16()
                for d in range(D):
                    vals = plsc.load_gather(rows_v, [rows16, colbase + d])
                    outb_v[d_off + d, pl.ds(L * v, L)] = vals

        dotable(iu_hbm, uidx_v, 0)
        dotable(ii_hbm, iidx_v, D)
        pltpu.sync_copy(outb_v, out_hbm.at[:, pl.ds(base, b_per_w)])

    return kb


def kernel(user_embedding, item_embedding, user_idx, item_idx):
    B = user_idx.shape[0]
    V, D = user_embedding.shape
    ut3 = user_embedding.T.reshape(D // 8, 8, V)
    it3 = item_embedding.T.reshape(D // 8, 8, V)
    inter_u, inter_i = _make_phase_a(V, D)(ut3, it3)
    out_t = _make_phase_b(B, D, inter_u.shape[0])(
        inter_u, inter_i,
        user_idx.astype(jnp.int32), item_idx.astype(jnp.int32))
    return out_t.T


# speedup vs baseline: 1.9521x; 1.9521x over previous
import functools

import jax
import jax.numpy as jnp
from jax import lax
from jax.experimental import pallas as pl
from jax.experimental.pallas import tpu as pltpu
from jax.experimental.pallas import tpu_sc as plsc

NC = 2
NS = 16
NW = NC * NS
CHUNK = 128
L = 16


def _iota16():
    return lax.iota(jnp.int32, L)


@functools.lru_cache(maxsize=None)
def _make_phase_a(V, D):
    n_tc = (V + CHUNK - 1) // CHUNK
    n_per_tile = (n_tc + NS - 1) // NS
    ri = n_tc * (CHUNK // 4)
    mesh = plsc.VectorSubcoreMesh(core_axis_name="c", subcore_axis_name="s")

    @functools.partial(
        pl.kernel,
        mesh=mesh,
        out_type=(jax.ShapeDtypeStruct((ri, 128), jnp.float32),
                  jax.ShapeDtypeStruct((ri, 128), jnp.float32)),
        scratch_types=[
            pltpu.VMEM((D, CHUNK), jnp.float32),
            pltpu.VMEM((CHUNK // 4, 128), jnp.float32),
            pltpu.SemaphoreType.DMA,
        ],
        compiler_params=pltpu.CompilerParams(disable_bounds_checks=True, needs_layout_passes=False),
    )
    def ka(ut3_hbm, it3_hbm, iu_hbm, ii_hbm, blk_v, outb_v, sem):
        cid = lax.axis_index("c")
        sid = lax.axis_index("s")

        def run(t3, inter):
            @pl.loop(0, n_per_tile)
            def _(t):
                tc = sid + t * NS

                @pl.when(tc < n_tc)
                def _():
                    col0 = pl.multiple_of(tc * CHUNK, CHUNK)
                    cps = [pltpu.async_copy(
                        t3.at[tr, :, pl.ds(col0, CHUNK)],
                        blk_v.at[pl.ds(8 * tr, 8)], sem)
                        for tr in range(D // 8)]
                    for cp in cps:
                        cp.wait()
                    rowbase = _iota16() // 4
                    colbase = (_iota16() % 4) * D
                    for c in range(D):
                        for m in range(CHUNK // L):
                            x = blk_v[c, pl.ds(L * m, L)]
                            plsc.store_scatter(
                                outb_v, [4 * m + rowbase, colbase + c], x)
                    row0 = pl.multiple_of(tc * (CHUNK // 4), CHUNK // 4)
                    pltpu.sync_copy(outb_v, inter.at[pl.ds(row0, CHUNK // 4)])

        @pl.when(cid == 0)
        def _():
            run(ut3_hbm, iu_hbm)

        @pl.when(cid == 1)
        def _():
            run(it3_hbm, ii_hbm)

    return ka


@functools.lru_cache(maxsize=None)
def _make_phase_b(B, D, ri):
    b_per_w = B // NW
    n_chunks = b_per_w // CHUNK
    mesh = plsc.VectorSubcoreMesh(core_axis_name="c", subcore_axis_name="s")

    @functools.partial(
        pl.kernel,
        mesh=mesh,
        out_type=jax.ShapeDtypeStruct((2 * D, B), jnp.float32),
        scratch_types=[
            pltpu.VMEM((b_per_w,), jnp.int32),
            pltpu.VMEM((b_per_w,), jnp.int32),
            pltpu.VMEM((b_per_w,), jnp.int32),
            pltpu.VMEM((b_per_w, 128), jnp.float32),
            pltpu.VMEM((2 * D, b_per_w), jnp.float32),
            pltpu.SemaphoreType.DMA,
        ],
        compiler_params=pltpu.CompilerParams(disable_bounds_checks=True, needs_layout_passes=False),
    )
    def kb(iu_hbm, ii_hbm, uidx_hbm, iidx_hbm, out_hbm,
           uidx_v, iidx_v, gr_v, rows_v, outb_v, sem):
        cid = lax.axis_index("c")
        sid = lax.axis_index("s")
        w = sid * NC + cid
        base = w * b_per_w
        pltpu.sync_copy(uidx_hbm.at[pl.ds(base, b_per_w)], uidx_v)
        pltpu.sync_copy(iidx_hbm.at[pl.ds(base, b_per_w)], iidx_v)

        def dotable(inter, idx_v, d_off):
            for v in range(b_per_w // L):
                x = idx_v[pl.ds(L * v, L)]
                gr_v[pl.ds(L * v, L)] = lax.shift_right_logical(x, 2)
            cps = []
            for j in range(n_chunks):
                cps.append(pltpu.async_copy(
                    inter.at[gr_v.at[pl.ds(j * CHUNK, CHUNK)]],
                    rows_v.at[pl.ds(j * CHUNK, CHUNK)], sem))
            for cp in cps:
                cp.wait()
            for v in range(b_per_w // L):
                x = idx_v[pl.ds(L * v, L)]
                colbase = (x & 3) * D
                rows16 = L * v + _iota16()
                for d in range(D):
                    vals = plsc.load_gather(rows_v, [rows16, colbase + d])
                    outb_v[d_off + d, pl.ds(L * v, L)] = vals

        dotable(iu_hbm, uidx_v, 0)
        dotable(ii_hbm, iidx_v, D)
        pltpu.sync_copy(outb_v, out_hbm.at[:, pl.ds(base, b_per_w)])

    return kb


def kernel(user_embedding, item_embedding, user_idx, item_idx):
    B = user_idx.shape[0]
    V, D = user_embedding.shape
    inter_u = user_embedding.reshape(V // 4, 4 * D)
    inter_i = item_embedding.reshape(V // 4, 4 * D)
    out_t = _make_phase_b(B, D, inter_u.shape[0])(
        inter_u, inter_i,
        user_idx.astype(jnp.int32), item_idx.astype(jnp.int32))
    return out_t.T


# speedup vs baseline: 2.0037x; 1.0264x over previous
import functools

import jax
import jax.numpy as jnp
from jax import lax
from jax.experimental import pallas as pl
from jax.experimental.pallas import tpu as pltpu
from jax.experimental.pallas import tpu_sc as plsc

NC = 2
NS = 16
NW = NC * NS
CHUNK = 128


@functools.lru_cache(maxsize=None)
def _make_kernel(B, D):
    b_per_w = B // NW
    n_chunks = b_per_w // CHUNK
    mesh = plsc.VectorSubcoreMesh(core_axis_name="c", subcore_axis_name="s")

    @functools.partial(
        pl.kernel,
        mesh=mesh,
        out_type=jax.ShapeDtypeStruct((B, 2 * D), jnp.float32),
        scratch_types=[
            pltpu.VMEM((b_per_w,), jnp.int32),
            pltpu.VMEM((b_per_w,), jnp.int32),
            pltpu.VMEM((b_per_w, D), jnp.float32),
            pltpu.VMEM((b_per_w, D), jnp.float32),
            pltpu.SemaphoreType.DMA,
            pltpu.SemaphoreType.DMA,
        ],
        compiler_params=pltpu.CompilerParams(use_tc_tiling_on_sc=False),
    )
    def k(user_hbm, item_hbm, uidx_hbm, iidx_hbm, out_hbm,
          uidx_v, iidx_v, urows_v, irows_v, usem, isem):
        wid = lax.axis_index("s") * NC + lax.axis_index("c")
        base = wid * b_per_w
        pltpu.sync_copy(uidx_hbm.at[pl.ds(base, b_per_w)], uidx_v)
        pltpu.sync_copy(iidx_hbm.at[pl.ds(base, b_per_w)], iidx_v)
        copies = []
        for j in range(n_chunks):
            uidx = uidx_v.at[pl.ds(j * CHUNK, CHUNK)]
            iidx = iidx_v.at[pl.ds(j * CHUNK, CHUNK)]
            copies.append(pltpu.async_copy(
                user_hbm.at[uidx],
                urows_v.at[pl.ds(j * CHUNK, CHUNK)], usem))
            copies.append(pltpu.async_copy(
                item_hbm.at[iidx],
                irows_v.at[pl.ds(j * CHUNK, CHUNK)], isem))
        for c in copies:
            c.wait()
        pltpu.sync_copy(urows_v, out_hbm.at[pl.ds(base, b_per_w), pl.ds(0, D)])
        pltpu.sync_copy(irows_v, out_hbm.at[pl.ds(base, b_per_w), pl.ds(D, D)])

    return k


def kernel(user_embedding, item_embedding, user_idx, item_idx):
    B = user_idx.shape[0]
    D = user_embedding.shape[1]
    return _make_kernel(B, D)(
        user_embedding, item_embedding,
        user_idx.astype(jnp.int32), item_idx.astype(jnp.int32))
